# baseline (speedup 1.0000x reference)
import jax
import jax.numpy as jnp
from jax import lax
from jax.experimental import pallas as pl

N = 4096
CUTOFF_SQ = 0.25 * 0.25
BR = 256


def _pair_kernel(xi_ref, xt4_ref, wb_ref, out_ref):
    w = wb_ref[0, 0]
    b = wb_ref[0, 1]
    dsq = None
    for k in range(3):
        xj = xt4_ref[k:k + 1]
        xi = xi_ref[:, k:k + 1][..., None, None]
        d = xj - xi
        d = d - jnp.round(d)
        dsq = d * d if dsq is None else dsq + d * d
    bits = lax.bitcast_convert_type(dsq, jnp.uint32)
    mask = (bits - jnp.uint32(1)) < jnp.uint32(0x3D800000 - 1)
    r = dsq * lax.rsqrt(dsq)
    out_ref[...] = jnp.where(mask, r * w + b, 0.0)


def kernel(xyz, W, b):
    xt4 = xyz.T.reshape(3, 4, 8, 128)
    wb = jnp.concatenate([W.reshape(1, 1), b.reshape(1, 1)], axis=1)
    out = pl.pallas_call(
        _pair_kernel,
        grid=(N // BR,),
        in_specs=[
            pl.BlockSpec((BR, 3), lambda i: (i, 0)),
            pl.BlockSpec((3, 4, 8, 128), lambda i: (0, 0, 0, 0)),
            pl.BlockSpec((1, 2), lambda i: (0, 0)),
        ],
        out_specs=pl.BlockSpec((BR, 4, 8, 128), lambda i: (i, 0, 0, 0)),
        out_shape=jax.ShapeDtypeStruct((N, 4, 8, 128), jnp.float32),
    )(xyz, xt4, wb)
    return jax.lax.reshape(out, (N, N, 1))
